# Initial kernel scaffold; baseline (speedup 1.0000x reference)
import jax
import jax.numpy as jnp
from jax.experimental import pallas as pl


def kernel(x, edge_index, W_ft, b_ft, W_s1, b_s1, W_s2, b_s2, W_g0, b_g0, W_g1, b_g1, W_a1, b_a1, W_a2, b_a2):
    raise NotImplementedError("write your pallas kernel here")


# speedup vs baseline: 7.8991x; 7.8991x over previous
import functools

import jax
import jax.numpy as jnp
from jax import lax
from jax.experimental import pallas as pl
from jax.experimental.pallas import tpu as pltpu
from jax.experimental.pallas import tpu_sc as plsc

N = 10000
E = 320000
D = 128
NP = 10240
NW = 32
EW = E // NW
K = 80
NCHUNK = EW // K
PT = NP // 16
SPL = 5120
ACR = 5248
APT = ACR // 16
DUM = ACR - 1
RB = 20
BR = NP // RB


def _mesh():
    return plsc.VectorSubcoreMesh(core_axis_name="c", subcore_axis_name="s")


def _round_bf16(z):
    u = plsc.bitcast(z, jnp.uint32)
    r = (u + jnp.uint32(0x7FFF) + ((u >> jnp.uint32(16)) & jnp.uint32(1))) \
        & jnp.uint32(0xFFFF0000)
    return plsc.bitcast(r, jnp.float32)


def _tca_body(x_ref, wft_ref, bft_ref, ws1_ref, bs1_ref, wg0_ref,
              ht_ref, a_ref, b_ref, g0_ref):
    ht = jnp.maximum(
        jnp.dot(x_ref[...], wft_ref[...], preferred_element_type=jnp.float32)
        + bft_ref[...], 0.0)
    ht_ref[...] = ht
    a_ref[...] = jnp.dot(ht, ws1_ref[0:D, :], preferred_element_type=jnp.float32)
    b_ref[...] = (jnp.dot(ht, ws1_ref[D:2 * D, :], preferred_element_type=jnp.float32)
                  + bs1_ref[...])
    g0_ref[...] = jnp.dot(ht, wg0_ref[...], preferred_element_type=jnp.float32)


def _tc_a(xp, W_ft, b_ft2, W_s1, b_s12, W_g0):
    blk = pl.BlockSpec((BR, D), lambda i: (i, 0))
    full = lambda s: pl.BlockSpec(s, lambda i: (0, 0))
    out = jax.ShapeDtypeStruct((NP, D), jnp.float32)
    return pl.pallas_call(
        _tca_body,
        grid=(RB,),
        in_specs=[blk, full((D, D)), full((1, D)), full((2 * D, D)),
                  full((1, D)), full((D, D))],
        out_specs=[blk, blk, blk, blk],
        out_shape=[out, out, out, out],
    )(xp, W_ft, b_ft2, W_s1, b_s12, W_g0)


def _thr_body(t_ref, out_ref):
    v = 1.0 / (1.0 + jnp.exp(-t_ref[...]))
    s1 = jnp.sum(v)
    s2 = jnp.sum(v * v)
    m = s1 / E
    var = (s2 - s1 * s1 / E) / (E - 1)
    thr = m + jnp.sqrt(var)
    tl = jnp.where(thr < 1.0, jnp.log(thr / (1.0 - thr)), jnp.float32(3.0e38))
    out_ref[...] = jnp.full((8, 128), tl, jnp.float32)


def _tc_thr(ew2d):
    return pl.pallas_call(
        _thr_body,
        out_shape=jax.ShapeDtypeStruct((8, 128), jnp.float32),
    )(ew2d)


def _scale_body(d0_ref, d1_ref, g0_ref, disb_ref, g0p_ref):
    d = d0_ref[0, 0, :] + d1_ref[0, 0, :] + 2.0
    dis = lax.rsqrt(d)
    disb = jnp.broadcast_to(dis[:, None], (BR, D))
    disb_ref[...] = disb
    g0p_ref[...] = disb * g0_ref[...]


def _tc_scale(d0, d1, g0_pre):
    blk = pl.BlockSpec((BR, D), lambda i: (i, 0))
    dblk = pl.BlockSpec((1, 1, BR), lambda i: (i, 0, 0))
    out = jax.ShapeDtypeStruct((NP, D), jnp.float32)
    return pl.pallas_call(
        _scale_body,
        grid=(RB,),
        in_specs=[dblk, dblk, blk],
        out_specs=[blk, blk],
        out_shape=[out, out],
    )(d0, d1, g0_pre)


def _layer_body(p0_ref, p1_ref, gp_ref, disb_ref, bg_ref, wg_ref, out_ref):
    h = jnp.maximum(
        disb_ref[...] * (p0_ref[...] + p1_ref[...] + 2.0 * gp_ref[...])
        + bg_ref[...], 0.0)
    out_ref[...] = disb_ref[...] * jnp.dot(h, wg_ref[...],
                                           preferred_element_type=jnp.float32)


def _tc_layer(p0, p1, gp, disb, bg2, Wg):
    blk = pl.BlockSpec((BR, D), lambda i: (i, 0))
    full = lambda s: pl.BlockSpec(s, lambda i: (0, 0))
    return pl.pallas_call(
        _layer_body,
        grid=(RB,),
        in_specs=[blk, blk, blk, blk, full((1, D)), full((D, D))],
        out_specs=blk,
        out_shape=jax.ShapeDtypeStruct((NP, D), jnp.float32),
    )(p0, p1, gp, disb, bg2, Wg)


def _final_body(q0_ref, q1_ref, gp_ref, disb_ref, bg_ref, ht_ref,
                wa1_ref, ba1_ref, wa2_ref, ba2_ref, out_ref):
    h2 = jnp.maximum(
        disb_ref[...] * (q0_ref[...] + q1_ref[...] + 2.0 * gp_ref[...])
        + bg_ref[...], 0.0)
    wa_h = wa1_ref[0:D, :] + wa1_ref[D:2 * D, :]
    pre = jnp.maximum(
        jnp.dot(ht_ref[...], wa_h, preferred_element_type=jnp.float32)
        + jnp.dot(h2, wa1_ref[2 * D:3 * D, :], preferred_element_type=jnp.float32)
        + ba1_ref[...], 0.0)
    out_ref[...] = (jnp.dot(pre, wa2_ref[...], preferred_element_type=jnp.float32)
                    + ba2_ref[...])


def _tc_final(q0, q1, g1p, disb, bg2, ht, Wa1, ba12, Wa2p, ba2p):
    blk = pl.BlockSpec((BR, D), lambda i: (i, 0))
    full = lambda s: pl.BlockSpec(s, lambda i: (0, 0))
    return pl.pallas_call(
        _final_body,
        grid=(RB,),
        in_specs=[blk, blk, blk, blk, full((1, D)), blk,
                  full((3 * D, D)), full((1, D)), full((D, D)), full((1, D))],
        out_specs=blk,
        out_shape=jax.ShapeDtypeStruct((NP, D), jnp.float32),
    )(q0, q1, g1p, disb, bg2, ht, Wa1, ba12, Wa2p, ba2p)


def _sc_sim(A, B, row_r, col_r, w2_8x16, b2_16):
    @functools.partial(
        pl.kernel,
        out_type=jax.ShapeDtypeStruct((NW, NCHUNK, K), jnp.float32),
        mesh=_mesh(),
        compiler_params=pltpu.CompilerParams(needs_layout_passes=False),
        scratch_types=[
            pltpu.VMEM((NCHUNK, K), jnp.int32),
            pltpu.VMEM((NCHUNK, K), jnp.int32),
            pltpu.VMEM((K, D), jnp.float32),
            pltpu.VMEM((K, D), jnp.float32),
            pltpu.VMEM((NCHUNK, K), jnp.float32),
            pltpu.VMEM((8, 16), jnp.float32),
            pltpu.VMEM((16,), jnp.float32),
            pltpu.SemaphoreType.DMA,
            pltpu.SemaphoreType.DMA,
        ],
    )
    def k(a_hbm, b_hbm, row_hbm, col_hbm, w2_hbm, b2_hbm, ew_hbm,
          idxr, idxc, abuf, bbuf, ewbuf, w2v, b2v, sem1, sem2):
        wid = lax.axis_index("s") * 2 + lax.axis_index("c")
        pltpu.sync_copy(row_hbm.at[wid], idxr)
        pltpu.sync_copy(col_hbm.at[wid], idxc)
        pltpu.sync_copy(w2_hbm, w2v)
        pltpu.sync_copy(b2_hbm, b2v)
        w2l = [_round_bf16(w2v[u, :]) for u in range(8)]
        b2 = b2v[...]
        lanes = lax.iota(jnp.int32, 16)

        def chunk(c, _):
            ca = pltpu.async_copy(a_hbm.at[idxr.at[c]], abuf, sem1)
            cb = pltpu.async_copy(b_hbm.at[idxc.at[c]], bbuf, sem2)
            ca.wait()
            cb.wait()

            def group(g, _):
                tot = jnp.zeros((16,), jnp.float32)
                for j16 in range(16):
                    j = g * 16 + j16
                    acc = jnp.zeros((16,), jnp.float32)
                    for u in range(8):
                        av = abuf[j, u * 16:(u + 1) * 16]
                        bv = bbuf[j, u * 16:(u + 1) * 16]
                        z = jnp.maximum(av + bv, 0.0)
                        acc = acc + _round_bf16(z) * w2l[u]
                    tot = jnp.where(lanes == j16, jnp.sum(acc), tot)
                ewbuf[c, pl.ds(g * 16, 16)] = tot + b2
                return 0

            lax.fori_loop(0, 5, group, 0)
            return 0

        lax.fori_loop(0, NCHUNK, chunk, 0)
        pltpu.sync_copy(ewbuf, ew_hbm.at[wid])

    return k(A, B, row_r, col_r, w2_8x16, b2_16)


def _sc_mask_deg(ew, col_r, thr16):
    @functools.partial(
        pl.kernel,
        out_type=(jax.ShapeDtypeStruct((NW, NCHUNK, K), jnp.int32),
                  jax.ShapeDtypeStruct((NW, NCHUNK, K), jnp.int32),
                  jax.ShapeDtypeStruct((2, NP), jnp.float32)),
        mesh=_mesh(),
        compiler_params=pltpu.CompilerParams(needs_layout_passes=False),
        scratch_types=[
            pltpu.VMEM((NCHUNK, K), jnp.float32),
            pltpu.VMEM((NCHUNK, K), jnp.int32),
            pltpu.VMEM((NCHUNK, K), jnp.int32),
            pltpu.VMEM((NCHUNK, K), jnp.int32),
            pltpu.VMEM((NCHUNK, K), jnp.float32),
            pltpu.VMEM((16,), jnp.float32),
            pltpu.VMEM((PT,), jnp.float32),
            pltpu.VMEM_SHARED((NP,), jnp.float32),
        ],
    )
    def k(ew_hbm, col_hbm, thr_hbm, colma_hbm, colmb_hbm, degp_hbm,
          ewbuf, colbuf, cmbufA, cmbufB, mbuf, thrv, zbuf, degacc):
        cid = lax.axis_index("c")
        sid = lax.axis_index("s")
        wid = sid * 2 + cid
        pltpu.sync_copy(ew_hbm.at[wid], ewbuf)
        pltpu.sync_copy(col_hbm.at[wid], colbuf)
        pltpu.sync_copy(thr_hbm, thrv)
        for t in range(PT // 16):
            zbuf[t * 16:(t + 1) * 16] = jnp.zeros((16,), jnp.float32)
        pltpu.sync_copy(zbuf, degacc.at[pl.ds(sid * PT, PT)])
        plsc.subcore_barrier()
        thr = thrv[...]
        dummy = jnp.full((16,), DUM, jnp.int32)

        def chunk(c, _):
            for g in range(K // 16):
                ewv = ewbuf[c, pl.ds(g * 16, 16)]
                cv = colbuf[c, pl.ds(g * 16, 16)]
                m = ewv > thr
                mbuf[c, pl.ds(g * 16, 16)] = jnp.where(m, 1.0, 0.0)
                cmbufA[c, pl.ds(g * 16, 16)] = jnp.where(
                    m & (cv < SPL), cv, dummy)
                cmbufB[c, pl.ds(g * 16, 16)] = jnp.where(
                    m & (cv >= SPL), cv - SPL, dummy)
            pltpu.sync_copy(mbuf.at[c], degacc.at[colbuf.at[c]], add=True)
            return 0

        lax.fori_loop(0, NCHUNK, chunk, 0)
        plsc.subcore_barrier()
        pltpu.sync_copy(cmbufA, colma_hbm.at[wid])
        pltpu.sync_copy(cmbufB, colmb_hbm.at[wid])
        pltpu.sync_copy(degacc.at[pl.ds(sid * PT, PT)],
                        degp_hbm.at[cid, pl.ds(sid * PT, PT)])

    return k(ew, col_r, thr16)


def _sc_agg_range(gp, row_r, colm_r):
    @functools.partial(
        pl.kernel,
        out_type=jax.ShapeDtypeStruct((2, ACR, D), jnp.float32),
        mesh=_mesh(),
        compiler_params=pltpu.CompilerParams(needs_layout_passes=False),
        scratch_types=[
            pltpu.VMEM((NCHUNK, K), jnp.int32),
            pltpu.VMEM((NCHUNK, K), jnp.int32),
            pltpu.VMEM((K, D), jnp.float32),
            pltpu.VMEM((41, D), jnp.float32),
            pltpu.VMEM_SHARED((ACR, D), jnp.float32),
            pltpu.SemaphoreType.DMA,
        ],
    )
    def k(g_hbm, row_hbm, colm_hbm, part_hbm, idxr, idxc, gbuf, zbuf, acc, sem):
        cid = lax.axis_index("c")
        sid = lax.axis_index("s")
        wid = sid * 2 + cid
        pltpu.sync_copy(row_hbm.at[wid], idxr)
        pltpu.sync_copy(colm_hbm.at[wid], idxc)
        for r in range(41):
            for u in range(8):
                zbuf[r, u * 16:(u + 1) * 16] = jnp.zeros((16,), jnp.float32)
        for b in range(APT // 41):
            pltpu.sync_copy(zbuf, acc.at[pl.ds(sid * APT + b * 41, 41)])
        plsc.subcore_barrier()

        def chunk(c, _):
            pltpu.async_copy(g_hbm.at[idxr.at[c]], gbuf, sem).wait()
            pltpu.sync_copy(gbuf, acc.at[idxc.at[c]], add=True)
            return 0

        lax.fori_loop(0, NCHUNK, chunk, 0)
        plsc.subcore_barrier()
        pltpu.sync_copy(acc.at[pl.ds(sid * APT, APT)],
                        part_hbm.at[cid, pl.ds(sid * APT, APT)])

    return k(gp, row_r, colm_r)


def _sc_agg(gp, row_r, colmA, colmB):
    pA = _sc_agg_range(gp, row_r, colmA)
    pB = _sc_agg_range(gp, row_r, colmB)
    return jnp.concatenate(
        [pA[:, :SPL], pB[:, :NP - SPL]], axis=1)


def kernel(x, edge_index, W_ft, b_ft, W_s1, b_s1, W_s2, b_s2,
           W_g0, b_g0, W_g1, b_g1, W_a1, b_a1, W_a2, b_a2):
    row_r = edge_index[0].reshape(NW, NCHUNK, K)
    col_r = edge_index[1].reshape(NW, NCHUNK, K)
    xp = jnp.pad(x, ((0, NP - N), (0, 0)))

    ht, A, B, g0_pre = _tc_a(xp, W_ft, b_ft.reshape(1, D), W_s1,
                             b_s1.reshape(1, D), W_g0)

    ew = _sc_sim(A, B, row_r, col_r,
                 W_s2[:, 0].reshape(8, 16), jnp.broadcast_to(b_s2, (16,)))

    thr_tile = _tc_thr(ew.reshape(E // 128, 128))
    thr16 = thr_tile[0, :16]

    colmA, colmB, deg_p = _sc_mask_deg(ew, col_r, thr16)

    d0 = deg_p[0].reshape(RB, 1, BR)
    d1 = deg_p[1].reshape(RB, 1, BR)
    disb, g0p = _tc_scale(d0, d1, g0_pre)

    p = _sc_agg(g0p, row_r, colmA, colmB)
    g1p = _tc_layer(p[0], p[1], g0p, disb, b_g0.reshape(1, D), W_g1)

    q = _sc_agg(g1p, row_r, colmA, colmB)

    Wa2p = jnp.pad(W_a2, ((0, 0), (0, D - 1)))
    ba2p = jnp.pad(b_a2, (0, D - 1)).reshape(1, D)
    scores = _tc_final(q[0], q[1], g1p, disb, b_g1.reshape(1, D), ht,
                       W_a1, b_a1.reshape(1, D), Wa2p, ba2p)
    return scores[:N, 0]


# speedup vs baseline: 10.7515x; 1.3611x over previous
import functools

import jax
import jax.numpy as jnp
from jax import lax
from jax.experimental import pallas as pl
from jax.experimental.pallas import tpu as pltpu
from jax.experimental.pallas import tpu_sc as plsc

N = 10000
E = 320000
D = 128
NP = 10240
NW = 32
EW = E // NW
K = 80
NCHUNK = EW // K
PT = NP // 16
SPL = 5120
ACR = 5248
APT = ACR // 16
DUM = ACR - 1
RB = 20
BR = NP // RB


def _mesh():
    return plsc.VectorSubcoreMesh(core_axis_name="c", subcore_axis_name="s")


def _round_bf16(z):
    u = plsc.bitcast(z, jnp.uint32)
    r = (u + jnp.uint32(0x7FFF) + ((u >> jnp.uint32(16)) & jnp.uint32(1))) \
        & jnp.uint32(0xFFFF0000)
    return plsc.bitcast(r, jnp.float32)


def _tca_body(x_ref, wft_ref, bft_ref, ws1_ref, bs1_ref, wg0_ref,
              ht_ref, a_ref, b_ref, g0_ref):
    ht = jnp.maximum(
        jnp.dot(x_ref[...], wft_ref[...], preferred_element_type=jnp.float32)
        + bft_ref[...], 0.0)
    ht_ref[...] = ht
    a_ref[...] = jnp.dot(ht, ws1_ref[0:D, :], preferred_element_type=jnp.float32)
    b_ref[...] = (jnp.dot(ht, ws1_ref[D:2 * D, :], preferred_element_type=jnp.float32)
                  + bs1_ref[...])
    g0_ref[...] = jnp.dot(ht, wg0_ref[...], preferred_element_type=jnp.float32)


def _tc_a(xp, W_ft, b_ft2, W_s1, b_s12, W_g0):
    blk = pl.BlockSpec((BR, D), lambda i: (i, 0))
    full = lambda s: pl.BlockSpec(s, lambda i: (0, 0))
    out = jax.ShapeDtypeStruct((NP, D), jnp.float32)
    return pl.pallas_call(
        _tca_body,
        grid=(RB,),
        in_specs=[blk, full((D, D)), full((1, D)), full((2 * D, D)),
                  full((1, D)), full((D, D))],
        out_specs=[blk, blk, blk, blk],
        out_shape=[out, out, out, out],
    )(xp, W_ft, b_ft2, W_s1, b_s12, W_g0)


def _thr_body(t_ref, out_ref):
    v = 1.0 / (1.0 + jnp.exp(-t_ref[...]))
    s1 = jnp.sum(v)
    s2 = jnp.sum(v * v)
    m = s1 / E
    var = (s2 - s1 * s1 / E) / (E - 1)
    thr = m + jnp.sqrt(var)
    tl = jnp.where(thr < 1.0, jnp.log(thr / (1.0 - thr)), jnp.float32(3.0e38))
    out_ref[...] = jnp.full((8, 128), tl, jnp.float32)


def _tc_thr(ew2d):
    return pl.pallas_call(
        _thr_body,
        out_shape=jax.ShapeDtypeStruct((8, 128), jnp.float32),
    )(ew2d)


def _scale_body(d0_ref, d1_ref, g0_ref, disb_ref, g0p_ref):
    d = d0_ref[0, 0, :] + d1_ref[0, 0, :] + 2.0
    dis = lax.rsqrt(d)
    disb = jnp.broadcast_to(dis[:, None], (BR, D))
    disb_ref[...] = disb
    g0p_ref[...] = disb * g0_ref[...]


def _tc_scale(d0, d1, g0_pre):
    blk = pl.BlockSpec((BR, D), lambda i: (i, 0))
    dblk = pl.BlockSpec((1, 1, BR), lambda i: (i, 0, 0))
    out = jax.ShapeDtypeStruct((NP, D), jnp.float32)
    return pl.pallas_call(
        _scale_body,
        grid=(RB,),
        in_specs=[dblk, dblk, blk],
        out_specs=[blk, blk],
        out_shape=[out, out],
    )(d0, d1, g0_pre)


def _layer_body(p0_ref, p1_ref, gp_ref, disb_ref, bg_ref, wg_ref, out_ref):
    h = jnp.maximum(
        disb_ref[...] * (p0_ref[...] + p1_ref[...] + 2.0 * gp_ref[...])
        + bg_ref[...], 0.0)
    out_ref[...] = disb_ref[...] * jnp.dot(h, wg_ref[...],
                                           preferred_element_type=jnp.float32)


def _tc_layer(p0, p1, gp, disb, bg2, Wg):
    blk = pl.BlockSpec((BR, D), lambda i: (i, 0))
    full = lambda s: pl.BlockSpec(s, lambda i: (0, 0))
    return pl.pallas_call(
        _layer_body,
        grid=(RB,),
        in_specs=[blk, blk, blk, blk, full((1, D)), full((D, D))],
        out_specs=blk,
        out_shape=jax.ShapeDtypeStruct((NP, D), jnp.float32),
    )(p0, p1, gp, disb, bg2, Wg)


def _final_body(q0_ref, q1_ref, gp_ref, disb_ref, bg_ref, ht_ref,
                wa1_ref, ba1_ref, wa2_ref, ba2_ref, out_ref):
    h2 = jnp.maximum(
        disb_ref[...] * (q0_ref[...] + q1_ref[...] + 2.0 * gp_ref[...])
        + bg_ref[...], 0.0)
    wa_h = wa1_ref[0:D, :] + wa1_ref[D:2 * D, :]
    pre = jnp.maximum(
        jnp.dot(ht_ref[...], wa_h, preferred_element_type=jnp.float32)
        + jnp.dot(h2, wa1_ref[2 * D:3 * D, :], preferred_element_type=jnp.float32)
        + ba1_ref[...], 0.0)
    out_ref[...] = (jnp.dot(pre, wa2_ref[...], preferred_element_type=jnp.float32)
                    + ba2_ref[...])


def _tc_final(q0, q1, g1p, disb, bg2, ht, Wa1, ba12, Wa2p, ba2p):
    blk = pl.BlockSpec((BR, D), lambda i: (i, 0))
    full = lambda s: pl.BlockSpec(s, lambda i: (0, 0))
    return pl.pallas_call(
        _final_body,
        grid=(RB,),
        in_specs=[blk, blk, blk, blk, full((1, D)), blk,
                  full((3 * D, D)), full((1, D)), full((D, D)), full((1, D))],
        out_specs=blk,
        out_shape=jax.ShapeDtypeStruct((NP, D), jnp.float32),
    )(q0, q1, g1p, disb, bg2, ht, Wa1, ba12, Wa2p, ba2p)


def _sc_sim(A, B, row_r, col_r, w2_8x16, b2_16):
    @functools.partial(
        pl.kernel,
        out_type=jax.ShapeDtypeStruct((NW, NCHUNK, K), jnp.float32),
        mesh=_mesh(),
        compiler_params=pltpu.CompilerParams(needs_layout_passes=False),
        scratch_types=[
            pltpu.VMEM((NCHUNK, K), jnp.int32),
            pltpu.VMEM((NCHUNK, K), jnp.int32),
            pltpu.VMEM((K, D), jnp.float32),
            pltpu.VMEM((K, D), jnp.float32),
            pltpu.VMEM((NCHUNK, K), jnp.float32),
            pltpu.VMEM((8, 16), jnp.float32),
            pltpu.VMEM((16,), jnp.float32),
            pltpu.SemaphoreType.DMA,
            pltpu.SemaphoreType.DMA,
        ],
    )
    def k(a_hbm, b_hbm, row_hbm, col_hbm, w2_hbm, b2_hbm, ew_hbm,
          idxr, idxc, abuf, bbuf, ewbuf, w2v, b2v, sem1, sem2):
        wid = lax.axis_index("s") * 2 + lax.axis_index("c")
        pltpu.sync_copy(row_hbm.at[wid], idxr)
        pltpu.sync_copy(col_hbm.at[wid], idxc)
        pltpu.sync_copy(w2_hbm, w2v)
        pltpu.sync_copy(b2_hbm, b2v)
        w2l = [_round_bf16(w2v[u, :]) for u in range(8)]
        b2 = b2v[...]
        lanes = lax.iota(jnp.int32, 16)

        def chunk(c, _):
            ca = pltpu.async_copy(a_hbm.at[idxr.at[c]], abuf, sem1)
            cb = pltpu.async_copy(b_hbm.at[idxc.at[c]], bbuf, sem2)
            ca.wait()
            cb.wait()

            def group(g, _):
                tot = jnp.zeros((16,), jnp.float32)
                for j16 in range(16):
                    j = g * 16 + j16
                    acc = jnp.zeros((16,), jnp.float32)
                    for u in range(8):
                        av = abuf[j, u * 16:(u + 1) * 16]
                        bv = bbuf[j, u * 16:(u + 1) * 16]
                        z = jnp.maximum(av + bv, 0.0)
                        acc = acc + _round_bf16(z) * w2l[u]
                    tot = jnp.where(lanes == j16, jnp.sum(acc), tot)
                ewbuf[c, pl.ds(g * 16, 16)] = tot + b2
                return 0

            lax.fori_loop(0, 5, group, 0)
            return 0

        lax.fori_loop(0, NCHUNK, chunk, 0)
        pltpu.sync_copy(ewbuf, ew_hbm.at[wid])

    return k(A, B, row_r, col_r, w2_8x16, b2_16)


KL = 128
CAP = EW // KL + 2


def _sc_mask_deg(ew, row_r, col_r, thr16):
    @functools.partial(
        pl.kernel,
        out_type=(jax.ShapeDtypeStruct((NW, CAP, KL), jnp.int32),
                  jax.ShapeDtypeStruct((NW, CAP, KL), jnp.int32),
                  jax.ShapeDtypeStruct((NW, CAP, KL), jnp.int32),
                  jax.ShapeDtypeStruct((NW, CAP, KL), jnp.int32),
                  jax.ShapeDtypeStruct((NW, 128), jnp.int32),
                  jax.ShapeDtypeStruct((NW, 128), jnp.int32),
                  jax.ShapeDtypeStruct((2, NP), jnp.float32)),
        mesh=_mesh(),
        compiler_params=pltpu.CompilerParams(needs_layout_passes=False),
        scratch_types=[
            pltpu.VMEM((NCHUNK, K), jnp.float32),
            pltpu.VMEM((NCHUNK, K), jnp.int32),
            pltpu.VMEM((NCHUNK, K), jnp.int32),
            pltpu.VMEM((NCHUNK, K), jnp.float32),
            pltpu.VMEM((CAP * KL,), jnp.int32),
            pltpu.VMEM((CAP * KL,), jnp.int32),
            pltpu.VMEM((CAP * KL,), jnp.int32),
            pltpu.VMEM((CAP * KL,), jnp.int32),
            pltpu.VMEM((16,), jnp.float32),
            pltpu.VMEM((128,), jnp.int32),
            pltpu.VMEM((PT,), jnp.float32),
            pltpu.VMEM_SHARED((NP,), jnp.float32),
        ],
    )
    def k(ew_hbm, row_hbm, col_hbm, thr_hbm,
          crowa_hbm, ccola_hbm, crowb_hbm, ccolb_hbm, cnta_hbm, cntb_hbm,
          degp_hbm,
          ewbuf, rowbuf, colbuf, mbuf, crA, ccA, crB, ccB, thrv, cntbuf,
          zbuf, degacc):
        cid = lax.axis_index("c")
        sid = lax.axis_index("s")
        wid = sid * 2 + cid
        pltpu.sync_copy(ew_hbm.at[wid], ewbuf)
        pltpu.sync_copy(row_hbm.at[wid], rowbuf)
        pltpu.sync_copy(col_hbm.at[wid], colbuf)
        pltpu.sync_copy(thr_hbm, thrv)
        for t in range(PT // 16):
            zbuf[t * 16:(t + 1) * 16] = jnp.zeros((16,), jnp.float32)
        pltpu.sync_copy(zbuf, degacc.at[pl.ds(sid * PT, PT)])
        plsc.subcore_barrier()
        thr = thrv[...]

        def chunk(c, offs):
            offA, offB = offs
            for g in range(K // 16):
                ewv = ewbuf[c, pl.ds(g * 16, 16)]
                cv = colbuf[c, pl.ds(g * 16, 16)]
                rv = rowbuf[c, pl.ds(g * 16, 16)]
                m = ewv > thr
                mbuf[c, pl.ds(g * 16, 16)] = jnp.where(m, 1.0, 0.0)
                mA = m & (cv < SPL)
                mB = m & (cv >= SPL)
                plsc.store_compressed(crA.at[pl.ds(offA, 16)], rv, mask=mA)
                plsc.store_compressed(ccA.at[pl.ds(offA, 16)], cv, mask=mA)
                plsc.store_compressed(crB.at[pl.ds(offB, 16)], rv, mask=mB)
                plsc.store_compressed(ccB.at[pl.ds(offB, 16)], cv - SPL, mask=mB)
                offA = offA + jnp.sum(mA.astype(jnp.int32))
                offB = offB + jnp.sum(mB.astype(jnp.int32))
            pltpu.sync_copy(mbuf.at[c], degacc.at[colbuf.at[c]], add=True)
            return (offA, offB)

        z0 = jnp.zeros((), jnp.int32)
        offA, offB = lax.fori_loop(0, NCHUNK, chunk, (z0, z0))
        zi16 = jnp.zeros((16,), jnp.int32)
        dum16 = jnp.full((16,), DUM, jnp.int32)
        for i in range(KL // 16):
            crA[pl.ds(offA + i * 16, 16)] = zi16
            ccA[pl.ds(offA + i * 16, 16)] = dum16
            crB[pl.ds(offB + i * 16, 16)] = zi16
            ccB[pl.ds(offB + i * 16, 16)] = dum16
        ncA = offA // KL + 1
        ncB = offB // KL + 1
        lanes = lax.iota(jnp.int32, 16)
        for i in range(8):
            cntbuf[i * 16:(i + 1) * 16] = jnp.where(
                lanes == 0, ncA, 0) if i == 0 else jnp.zeros((16,), jnp.int32)
        pltpu.sync_copy(cntbuf, cnta_hbm.at[wid])
        cntbuf[0:16] = jnp.where(lanes == 0, ncB, 0)
        pltpu.sync_copy(cntbuf, cntb_hbm.at[wid])

        def wrA(c, _):
            pltpu.sync_copy(crA.at[pl.ds(c * KL, KL)], crowa_hbm.at[wid, c])
            pltpu.sync_copy(ccA.at[pl.ds(c * KL, KL)], ccola_hbm.at[wid, c])
            return 0

        def wrB(c, _):
            pltpu.sync_copy(crB.at[pl.ds(c * KL, KL)], crowb_hbm.at[wid, c])
            pltpu.sync_copy(ccB.at[pl.ds(c * KL, KL)], ccolb_hbm.at[wid, c])
            return 0

        lax.fori_loop(0, ncA, wrA, 0)
        lax.fori_loop(0, ncB, wrB, 0)
        plsc.subcore_barrier()
        pltpu.sync_copy(degacc.at[pl.ds(sid * PT, PT)],
                        degp_hbm.at[cid, pl.ds(sid * PT, PT)])

    return k(ew, row_r, col_r, thr16)


def _sc_agg_range(gp, crow, ccol, cnt):
    @functools.partial(
        pl.kernel,
        out_type=jax.ShapeDtypeStruct((2, ACR, D), jnp.float32),
        mesh=_mesh(),
        compiler_params=pltpu.CompilerParams(needs_layout_passes=False),
        scratch_types=[
            pltpu.VMEM((CAP, KL), jnp.int32),
            pltpu.VMEM((CAP, KL), jnp.int32),
            pltpu.VMEM((KL, D), jnp.float32),
            pltpu.VMEM((128,), jnp.int32),
            pltpu.VMEM((41, D), jnp.float32),
            pltpu.VMEM_SHARED((ACR, D), jnp.float32),
            pltpu.SemaphoreType.DMA,
        ],
    )
    def k(g_hbm, crow_hbm, ccol_hbm, cnt_hbm, part_hbm,
          idxr, idxc, gbuf, cbuf, zbuf, acc, sem):
        cid = lax.axis_index("c")
        sid = lax.axis_index("s")
        wid = sid * 2 + cid
        pltpu.sync_copy(cnt_hbm.at[wid], cbuf)
        pltpu.sync_copy(crow_hbm.at[wid], idxr)
        pltpu.sync_copy(ccol_hbm.at[wid], idxc)
        for r in range(41):
            for u in range(8):
                zbuf[r, u * 16:(u + 1) * 16] = jnp.zeros((16,), jnp.float32)
        for b in range(APT // 41):
            pltpu.sync_copy(zbuf, acc.at[pl.ds(sid * APT + b * 41, 41)])
        nc = cbuf[0:16][0]
        plsc.subcore_barrier()

        def chunk(c, _):
            pltpu.async_copy(g_hbm.at[idxr.at[c]], gbuf, sem).wait()
            pltpu.sync_copy(gbuf, acc.at[idxc.at[c]], add=True)
            return 0

        lax.fori_loop(0, nc, chunk, 0)
        plsc.subcore_barrier()
        pltpu.sync_copy(acc.at[pl.ds(sid * APT, APT)],
                        part_hbm.at[cid, pl.ds(sid * APT, APT)])

    return k(gp, crow, ccol, cnt)


def _sc_agg(gp, lists):
    crowA, ccolA, crowB, ccolB, cntA, cntB = lists
    pA = _sc_agg_range(gp, crowA, ccolA, cntA)
    pB = _sc_agg_range(gp, crowB, ccolB, cntB)
    return jnp.concatenate(
        [pA[:, :SPL], pB[:, :NP - SPL]], axis=1)


def kernel(x, edge_index, W_ft, b_ft, W_s1, b_s1, W_s2, b_s2,
           W_g0, b_g0, W_g1, b_g1, W_a1, b_a1, W_a2, b_a2):
    row_r = edge_index[0].reshape(NW, NCHUNK, K)
    col_r = edge_index[1].reshape(NW, NCHUNK, K)
    xp = jnp.pad(x, ((0, NP - N), (0, 0)))

    ht, A, B, g0_pre = _tc_a(xp, W_ft, b_ft.reshape(1, D), W_s1,
                             b_s1.reshape(1, D), W_g0)

    ew = _sc_sim(A, B, row_r, col_r,
                 W_s2[:, 0].reshape(8, 16), jnp.broadcast_to(b_s2, (16,)))

    thr_tile = _tc_thr(ew.reshape(E // 128, 128))
    thr16 = thr_tile[0, :16]

    (crowA, ccolA, crowB, ccolB, cntA, cntB,
     deg_p) = _sc_mask_deg(ew, row_r, col_r, thr16)
    lists = (crowA, ccolA, crowB, ccolB, cntA, cntB)

    d0 = deg_p[0].reshape(RB, 1, BR)
    d1 = deg_p[1].reshape(RB, 1, BR)
    disb, g0p = _tc_scale(d0, d1, g0_pre)

    p = _sc_agg(g0p, lists)
    g1p = _tc_layer(p[0], p[1], g0p, disb, b_g0.reshape(1, D), W_g1)

    q = _sc_agg(g1p, lists)

    Wa2p = jnp.pad(W_a2, ((0, 0), (0, D - 1)))
    ba2p = jnp.pad(b_a2, (0, D - 1)).reshape(1, D)
    scores = _tc_final(q[0], q[1], g1p, disb, b_g1.reshape(1, D), ht,
                       W_a1, b_a1.reshape(1, D), Wa2p, ba2p)
    return scores[:N, 0]


# speedup vs baseline: 10.9852x; 1.0217x over previous
import functools

import jax
import jax.numpy as jnp
from jax import lax
from jax.experimental import pallas as pl
from jax.experimental.pallas import tpu as pltpu
from jax.experimental.pallas import tpu_sc as plsc

N = 10000
E = 320000
D = 128
NP = 10240
NW = 32
EW = E // NW
K = 80
NCHUNK = EW // K
PT = NP // 16
SPL = 5120
ACR = 5248
APT = ACR // 16
DUM = ACR - 1
RB = 20
BR = NP // RB


def _mesh():
    return plsc.VectorSubcoreMesh(core_axis_name="c", subcore_axis_name="s")


def _round_bf16(z):
    u = plsc.bitcast(z, jnp.uint32)
    r = (u + jnp.uint32(0x7FFF) + ((u >> jnp.uint32(16)) & jnp.uint32(1))) \
        & jnp.uint32(0xFFFF0000)
    return plsc.bitcast(r, jnp.float32)


def _tca_body(x_ref, wft_ref, bft_ref, ws1_ref, bs1_ref, wg0_ref,
              ht_ref, a_ref, b_ref, g0_ref):
    ht = jnp.maximum(
        jnp.dot(x_ref[...], wft_ref[...], preferred_element_type=jnp.float32)
        + bft_ref[...], 0.0)
    ht_ref[...] = ht
    a_ref[...] = jnp.dot(ht, ws1_ref[0:D, :], preferred_element_type=jnp.float32)
    b_ref[...] = (jnp.dot(ht, ws1_ref[D:2 * D, :], preferred_element_type=jnp.float32)
                  + bs1_ref[...])
    g0_ref[...] = jnp.dot(ht, wg0_ref[...], preferred_element_type=jnp.float32)


def _tc_a(xp, W_ft, b_ft2, W_s1, b_s12, W_g0):
    blk = pl.BlockSpec((BR, D), lambda i: (i, 0))
    full = lambda s: pl.BlockSpec(s, lambda i: (0, 0))
    out = jax.ShapeDtypeStruct((NP, D), jnp.float32)
    return pl.pallas_call(
        _tca_body,
        grid=(RB,),
        in_specs=[blk, full((D, D)), full((1, D)), full((2 * D, D)),
                  full((1, D)), full((D, D))],
        out_specs=[blk, blk, blk, blk],
        out_shape=[out, out, out, out],
    )(xp, W_ft, b_ft2, W_s1, b_s12, W_g0)


def _thr_body(t_ref, out_ref):
    v = 1.0 / (1.0 + jnp.exp(-t_ref[...]))
    s1 = jnp.sum(v)
    s2 = jnp.sum(v * v)
    m = s1 / E
    var = (s2 - s1 * s1 / E) / (E - 1)
    thr = m + jnp.sqrt(var)
    tl = jnp.where(thr < 1.0, jnp.log(thr / (1.0 - thr)), jnp.float32(3.0e38))
    out_ref[...] = jnp.full((8, 128), tl, jnp.float32)


def _tc_thr(ew2d):
    return pl.pallas_call(
        _thr_body,
        out_shape=jax.ShapeDtypeStruct((8, 128), jnp.float32),
    )(ew2d)


def _scale_body(d0_ref, d1_ref, g0_ref, disb_ref, g0p_ref):
    d = d0_ref[0, 0, :] + d1_ref[0, 0, :] + 2.0
    dis = lax.rsqrt(d)
    disb = jnp.broadcast_to(dis[:, None], (BR, D))
    disb_ref[...] = disb
    g0p_ref[...] = disb * g0_ref[...]


def _tc_scale(d0, d1, g0_pre):
    blk = pl.BlockSpec((BR, D), lambda i: (i, 0))
    dblk = pl.BlockSpec((1, 1, BR), lambda i: (i, 0, 0))
    out = jax.ShapeDtypeStruct((NP, D), jnp.float32)
    return pl.pallas_call(
        _scale_body,
        grid=(RB,),
        in_specs=[dblk, dblk, blk],
        out_specs=[blk, blk],
        out_shape=[out, out],
    )(d0, d1, g0_pre)


def _layer_body(p0_ref, p1_ref, gp_ref, disb_ref, bg_ref, wg_ref, out_ref):
    h = jnp.maximum(
        disb_ref[...] * (p0_ref[...] + p1_ref[...] + 2.0 * gp_ref[...])
        + bg_ref[...], 0.0)
    out_ref[...] = disb_ref[...] * jnp.dot(h, wg_ref[...],
                                           preferred_element_type=jnp.float32)


def _tc_layer(p0, p1, gp, disb, bg2, Wg):
    blk = pl.BlockSpec((BR, D), lambda i: (i, 0))
    full = lambda s: pl.BlockSpec(s, lambda i: (0, 0))
    return pl.pallas_call(
        _layer_body,
        grid=(RB,),
        in_specs=[blk, blk, blk, blk, full((1, D)), full((D, D))],
        out_specs=blk,
        out_shape=jax.ShapeDtypeStruct((NP, D), jnp.float32),
    )(p0, p1, gp, disb, bg2, Wg)


def _final_body(q0_ref, q1_ref, gp_ref, disb_ref, bg_ref, ht_ref,
                wa1_ref, ba1_ref, wa2_ref, ba2_ref, out_ref):
    h2 = jnp.maximum(
        disb_ref[...] * (q0_ref[...] + q1_ref[...] + 2.0 * gp_ref[...])
        + bg_ref[...], 0.0)
    wa_h = wa1_ref[0:D, :] + wa1_ref[D:2 * D, :]
    pre = jnp.maximum(
        jnp.dot(ht_ref[...], wa_h, preferred_element_type=jnp.float32)
        + jnp.dot(h2, wa1_ref[2 * D:3 * D, :], preferred_element_type=jnp.float32)
        + ba1_ref[...], 0.0)
    out_ref[...] = (jnp.dot(pre, wa2_ref[...], preferred_element_type=jnp.float32)
                    + ba2_ref[...])


def _tc_final(q0, q1, g1p, disb, bg2, ht, Wa1, ba12, Wa2p, ba2p):
    blk = pl.BlockSpec((BR, D), lambda i: (i, 0))
    full = lambda s: pl.BlockSpec(s, lambda i: (0, 0))
    return pl.pallas_call(
        _final_body,
        grid=(RB,),
        in_specs=[blk, blk, blk, blk, full((1, D)), blk,
                  full((3 * D, D)), full((1, D)), full((D, D)), full((1, D))],
        out_specs=blk,
        out_shape=jax.ShapeDtypeStruct((NP, D), jnp.float32),
    )(q0, q1, g1p, disb, bg2, ht, Wa1, ba12, Wa2p, ba2p)


def _sc_sim(A, B, row_r, col_r, w2_8x16, b2_16):
    @functools.partial(
        pl.kernel,
        out_type=jax.ShapeDtypeStruct((NW, NCHUNK, K), jnp.float32),
        mesh=_mesh(),
        compiler_params=pltpu.CompilerParams(needs_layout_passes=False),
        scratch_types=[
            pltpu.VMEM((NCHUNK, K), jnp.int32),
            pltpu.VMEM((NCHUNK, K), jnp.int32),
            pltpu.VMEM((K, D), jnp.float32),
            pltpu.VMEM((K, D), jnp.float32),
            pltpu.VMEM((NCHUNK, K), jnp.float32),
            pltpu.VMEM((8, 16), jnp.float32),
            pltpu.VMEM((16,), jnp.float32),
            pltpu.SemaphoreType.DMA,
            pltpu.SemaphoreType.DMA,
        ],
    )
    def k(a_hbm, b_hbm, row_hbm, col_hbm, w2_hbm, b2_hbm, ew_hbm,
          idxr, idxc, abuf, bbuf, ewbuf, w2v, b2v, sem1, sem2):
        wid = lax.axis_index("s") * 2 + lax.axis_index("c")
        pltpu.sync_copy(row_hbm.at[wid], idxr)
        pltpu.sync_copy(col_hbm.at[wid], idxc)
        pltpu.sync_copy(w2_hbm, w2v)
        pltpu.sync_copy(b2_hbm, b2v)
        w2l = [_round_bf16(w2v[u, :]) for u in range(8)]
        b2 = b2v[...]
        lanes = lax.iota(jnp.int32, 16)

        def chunk(c, _):
            ca = pltpu.async_copy(a_hbm.at[idxr.at[c]], abuf, sem1)
            cb = pltpu.async_copy(b_hbm.at[idxc.at[c]], bbuf, sem2)
            ca.wait()
            cb.wait()

            def group(g, _):
                tot = jnp.zeros((16,), jnp.float32)
                for j16 in range(16):
                    j = g * 16 + j16
                    acc = jnp.zeros((16,), jnp.float32)
                    for u in range(8):
                        av = abuf[j, u * 16:(u + 1) * 16]
                        bv = bbuf[j, u * 16:(u + 1) * 16]
                        z = jnp.maximum(av + bv, 0.0)
                        acc = acc + _round_bf16(z) * w2l[u]
                    tot = jnp.where(lanes == j16, jnp.sum(acc), tot)
                ewbuf[c, pl.ds(g * 16, 16)] = tot + b2
                return 0

            lax.fori_loop(0, 5, group, 0)
            return 0

        lax.fori_loop(0, NCHUNK, chunk, 0)
        pltpu.sync_copy(ewbuf, ew_hbm.at[wid])

    return k(A, B, row_r, col_r, w2_8x16, b2_16)


KL = 128
CAP = EW // KL + 2


def _sc_mask_deg(ew, row_r, col_r, thr16):
    @functools.partial(
        pl.kernel,
        out_type=(jax.ShapeDtypeStruct((NW, CAP, KL), jnp.int32),
                  jax.ShapeDtypeStruct((NW, CAP, KL), jnp.int32),
                  jax.ShapeDtypeStruct((NW, CAP, KL), jnp.int32),
                  jax.ShapeDtypeStruct((NW, CAP, KL), jnp.int32),
                  jax.ShapeDtypeStruct((NW, 128), jnp.int32),
                  jax.ShapeDtypeStruct((NW, 128), jnp.int32),
                  jax.ShapeDtypeStruct((2, NP), jnp.float32)),
        mesh=_mesh(),
        compiler_params=pltpu.CompilerParams(needs_layout_passes=False),
        scratch_types=[
            pltpu.VMEM((NCHUNK, K), jnp.float32),
            pltpu.VMEM((NCHUNK, K), jnp.int32),
            pltpu.VMEM((NCHUNK, K), jnp.int32),
            pltpu.VMEM((NCHUNK, K), jnp.float32),
            pltpu.VMEM((CAP * KL,), jnp.int32),
            pltpu.VMEM((CAP * KL,), jnp.int32),
            pltpu.VMEM((CAP * KL,), jnp.int32),
            pltpu.VMEM((CAP * KL,), jnp.int32),
            pltpu.VMEM((16,), jnp.float32),
            pltpu.VMEM((128,), jnp.int32),
            pltpu.VMEM((PT,), jnp.float32),
            pltpu.VMEM_SHARED((NP,), jnp.float32),
        ],
    )
    def k(ew_hbm, row_hbm, col_hbm, thr_hbm,
          crowa_hbm, ccola_hbm, crowb_hbm, ccolb_hbm, cnta_hbm, cntb_hbm,
          degp_hbm,
          ewbuf, rowbuf, colbuf, mbuf, crA, ccA, crB, ccB, thrv, cntbuf,
          zbuf, degacc):
        cid = lax.axis_index("c")
        sid = lax.axis_index("s")
        wid = sid * 2 + cid
        pltpu.sync_copy(ew_hbm.at[wid], ewbuf)
        pltpu.sync_copy(row_hbm.at[wid], rowbuf)
        pltpu.sync_copy(col_hbm.at[wid], colbuf)
        pltpu.sync_copy(thr_hbm, thrv)
        for t in range(PT // 16):
            zbuf[t * 16:(t + 1) * 16] = jnp.zeros((16,), jnp.float32)
        pltpu.sync_copy(zbuf, degacc.at[pl.ds(sid * PT, PT)])
        plsc.subcore_barrier()
        thr = thrv[...]

        def chunk(c, offs):
            offA, offB = offs
            for g in range(K // 16):
                ewv = ewbuf[c, pl.ds(g * 16, 16)]
                cv = colbuf[c, pl.ds(g * 16, 16)]
                rv = rowbuf[c, pl.ds(g * 16, 16)]
                m = ewv > thr
                mbuf[c, pl.ds(g * 16, 16)] = jnp.where(m, 1.0, 0.0)
                mA = m & (cv < SPL)
                mB = m & (cv >= SPL)
                plsc.store_compressed(crA.at[pl.ds(offA, 16)], rv, mask=mA)
                plsc.store_compressed(ccA.at[pl.ds(offA, 16)], cv, mask=mA)
                plsc.store_compressed(crB.at[pl.ds(offB, 16)], rv, mask=mB)
                plsc.store_compressed(ccB.at[pl.ds(offB, 16)], cv - SPL, mask=mB)
                offA = offA + jnp.sum(mA.astype(jnp.int32))
                offB = offB + jnp.sum(mB.astype(jnp.int32))
            pltpu.sync_copy(mbuf.at[c], degacc.at[colbuf.at[c]], add=True)
            return (offA, offB)

        z0 = jnp.zeros((), jnp.int32)
        offA, offB = lax.fori_loop(0, NCHUNK, chunk, (z0, z0))
        zi16 = jnp.zeros((16,), jnp.int32)
        dum16 = jnp.full((16,), DUM, jnp.int32)
        for i in range(KL // 16):
            crA[pl.ds(offA + i * 16, 16)] = zi16
            ccA[pl.ds(offA + i * 16, 16)] = dum16
            crB[pl.ds(offB + i * 16, 16)] = zi16
            ccB[pl.ds(offB + i * 16, 16)] = dum16
        ncA = offA // KL + 1
        ncB = offB // KL + 1
        lanes = lax.iota(jnp.int32, 16)
        for i in range(8):
            cntbuf[i * 16:(i + 1) * 16] = jnp.where(
                lanes == 0, ncA, 0) if i == 0 else jnp.zeros((16,), jnp.int32)
        pltpu.sync_copy(cntbuf, cnta_hbm.at[wid])
        cntbuf[0:16] = jnp.where(lanes == 0, ncB, 0)
        pltpu.sync_copy(cntbuf, cntb_hbm.at[wid])

        def wrA(c, _):
            pltpu.sync_copy(crA.at[pl.ds(c * KL, KL)], crowa_hbm.at[wid, c])
            pltpu.sync_copy(ccA.at[pl.ds(c * KL, KL)], ccola_hbm.at[wid, c])
            return 0

        def wrB(c, _):
            pltpu.sync_copy(crB.at[pl.ds(c * KL, KL)], crowb_hbm.at[wid, c])
            pltpu.sync_copy(ccB.at[pl.ds(c * KL, KL)], ccolb_hbm.at[wid, c])
            return 0

        lax.fori_loop(0, ncA, wrA, 0)
        lax.fori_loop(0, ncB, wrB, 0)
        plsc.subcore_barrier()
        pltpu.sync_copy(degacc.at[pl.ds(sid * PT, PT)],
                        degp_hbm.at[cid, pl.ds(sid * PT, PT)])

    return k(ew, row_r, col_r, thr16)


def _sc_agg_range(gp, crow, ccol, cnt):
    @functools.partial(
        pl.kernel,
        out_type=jax.ShapeDtypeStruct((2, ACR, D), jnp.float32),
        mesh=_mesh(),
        compiler_params=pltpu.CompilerParams(needs_layout_passes=False),
        scratch_types=[
            pltpu.VMEM((CAP, KL), jnp.int32),
            pltpu.VMEM((CAP, KL), jnp.int32),
            pltpu.VMEM((KL, D), jnp.float32),
            pltpu.VMEM((KL, D), jnp.float32),
            pltpu.VMEM((128,), jnp.int32),
            pltpu.VMEM((41, D), jnp.float32),
            pltpu.VMEM_SHARED((ACR, D), jnp.float32),
            pltpu.SemaphoreType.DMA,
            pltpu.SemaphoreType.DMA,
        ],
    )
    def k(g_hbm, crow_hbm, ccol_hbm, cnt_hbm, part_hbm,
          idxr, idxc, gbuf0, gbuf1, cbuf, zbuf, acc, sem0, sem1):
        cid = lax.axis_index("c")
        sid = lax.axis_index("s")
        wid = sid * 2 + cid
        pltpu.sync_copy(cnt_hbm.at[wid], cbuf)
        pltpu.sync_copy(crow_hbm.at[wid], idxr)
        pltpu.sync_copy(ccol_hbm.at[wid], idxc)
        for r in range(41):
            for u in range(8):
                zbuf[r, u * 16:(u + 1) * 16] = jnp.zeros((16,), jnp.float32)
        for b in range(APT // 41):
            pltpu.sync_copy(zbuf, acc.at[pl.ds(sid * APT + b * 41, 41)])
        nc = cbuf[0:16][0]
        plsc.subcore_barrier()
        bufs = (gbuf0, gbuf1)
        sems = (sem0, sem1)

        @pl.when(nc > 0)
        def _():
            pltpu.async_copy(g_hbm.at[idxr.at[0]], gbuf0, sem0)

        @pl.when(nc > 1)
        def _():
            pltpu.async_copy(g_hbm.at[idxr.at[1]], gbuf1, sem1)

        @pl.loop(0, nc, step=2)
        def _(c0):
            for b in range(2):
                c = c0 + b

                @pl.when(c < nc)
                def _():
                    pltpu.make_async_copy(
                        g_hbm.at[idxr.at[c]], bufs[b], sems[b]).wait()
                    pltpu.sync_copy(bufs[b], acc.at[idxc.at[c]], add=True)

                    @pl.when(c + 2 < nc)
                    def _():
                        pltpu.async_copy(
                            g_hbm.at[idxr.at[c + 2]], bufs[b], sems[b])

        plsc.subcore_barrier()
        pltpu.sync_copy(acc.at[pl.ds(sid * APT, APT)],
                        part_hbm.at[cid, pl.ds(sid * APT, APT)])

    return k(gp, crow, ccol, cnt)


def _sc_agg(gp, lists):
    crowA, ccolA, crowB, ccolB, cntA, cntB = lists
    pA = _sc_agg_range(gp, crowA, ccolA, cntA)
    pB = _sc_agg_range(gp, crowB, ccolB, cntB)
    return jnp.concatenate(
        [pA[:, :SPL], pB[:, :NP - SPL]], axis=1)


def kernel(x, edge_index, W_ft, b_ft, W_s1, b_s1, W_s2, b_s2,
           W_g0, b_g0, W_g1, b_g1, W_a1, b_a1, W_a2, b_a2):
    row_r = edge_index[0].reshape(NW, NCHUNK, K)
    col_r = edge_index[1].reshape(NW, NCHUNK, K)
    xp = jnp.pad(x, ((0, NP - N), (0, 0)))

    ht, A, B, g0_pre = _tc_a(xp, W_ft, b_ft.reshape(1, D), W_s1,
                             b_s1.reshape(1, D), W_g0)

    ew = _sc_sim(A, B, row_r, col_r,
                 W_s2[:, 0].reshape(8, 16), jnp.broadcast_to(b_s2, (16,)))

    thr_tile = _tc_thr(ew.reshape(E // 128, 128))
    thr16 = thr_tile[0, :16]

    (crowA, ccolA, crowB, ccolB, cntA, cntB,
     deg_p) = _sc_mask_deg(ew, row_r, col_r, thr16)
    lists = (crowA, ccolA, crowB, ccolB, cntA, cntB)

    d0 = deg_p[0].reshape(RB, 1, BR)
    d1 = deg_p[1].reshape(RB, 1, BR)
    disb, g0p = _tc_scale(d0, d1, g0_pre)

    p = _sc_agg(g0p, lists)
    g1p = _tc_layer(p[0], p[1], g0p, disb, b_g0.reshape(1, D), W_g1)

    q = _sc_agg(g1p, lists)

    Wa2p = jnp.pad(W_a2, ((0, 0), (0, D - 1)))
    ba2p = jnp.pad(b_a2, (0, D - 1)).reshape(1, D)
    scores = _tc_final(q[0], q[1], g1p, disb, b_g1.reshape(1, D), ht,
                       W_a1, b_a1.reshape(1, D), Wa2p, ba2p)
    return scores[:N, 0]


# speedup vs baseline: 13.0693x; 1.1897x over previous
import functools

import jax
import jax.numpy as jnp
from jax import lax
from jax.experimental import pallas as pl
from jax.experimental.pallas import tpu as pltpu
from jax.experimental.pallas import tpu_sc as plsc

N = 10000
E = 320000
D = 128
NP = 10240
NW = 32
EW = E // NW
K = 80
NCHUNK = EW // K
PT = NP // 16
SPL = 5120
ACR = 5248
APT = ACR // 16
DUM = ACR - 1
RB = 20
BR = NP // RB


def _mesh():
    return plsc.VectorSubcoreMesh(core_axis_name="c", subcore_axis_name="s")


def _round_bf16(z):
    u = plsc.bitcast(z, jnp.uint32)
    r = (u + jnp.uint32(0x7FFF) + ((u >> jnp.uint32(16)) & jnp.uint32(1))) \
        & jnp.uint32(0xFFFF0000)
    return plsc.bitcast(r, jnp.float32)


def _tca_body(x_ref, wft_ref, bft_ref, ws1_ref, bs1_ref, wg0_ref,
              ht_ref, a_ref, b_ref, g0_ref):
    ht = jnp.maximum(
        jnp.dot(x_ref[...], wft_ref[...], preferred_element_type=jnp.float32)
        + bft_ref[...], 0.0)
    ht_ref[...] = ht
    a_ref[...] = jnp.dot(ht, ws1_ref[0:D, :], preferred_element_type=jnp.float32)
    b_ref[...] = (jnp.dot(ht, ws1_ref[D:2 * D, :], preferred_element_type=jnp.float32)
                  + bs1_ref[...])
    g0_ref[...] = jnp.dot(ht, wg0_ref[...], preferred_element_type=jnp.float32)


def _tc_a(xp, W_ft, b_ft2, W_s1, b_s12, W_g0):
    blk = pl.BlockSpec((BR, D), lambda i: (i, 0))
    full = lambda s: pl.BlockSpec(s, lambda i: (0, 0))
    out = jax.ShapeDtypeStruct((NP, D), jnp.float32)
    return pl.pallas_call(
        _tca_body,
        grid=(RB,),
        in_specs=[blk, full((D, D)), full((1, D)), full((2 * D, D)),
                  full((1, D)), full((D, D))],
        out_specs=[blk, blk, blk, blk],
        out_shape=[out, out, out, out],
    )(xp, W_ft, b_ft2, W_s1, b_s12, W_g0)


def _thr_body(t_ref, out_ref):
    v = 1.0 / (1.0 + jnp.exp(-t_ref[...]))
    s1 = jnp.sum(v)
    s2 = jnp.sum(v * v)
    m = s1 / E
    var = (s2 - s1 * s1 / E) / (E - 1)
    thr = m + jnp.sqrt(var)
    tl = jnp.where(thr < 1.0, jnp.log(thr / (1.0 - thr)), jnp.float32(3.0e38))
    out_ref[...] = jnp.full((8, 128), tl, jnp.float32)


def _tc_thr(ew2d):
    return pl.pallas_call(
        _thr_body,
        out_shape=jax.ShapeDtypeStruct((8, 128), jnp.float32),
    )(ew2d)


def _scale_body(d0_ref, d1_ref, g0_ref, disb_ref, g0p_ref):
    d = d0_ref[0, 0, :] + d1_ref[0, 0, :] + 2.0
    dis = lax.rsqrt(d)
    disb = jnp.broadcast_to(dis[:, None], (BR, D))
    disb_ref[...] = disb
    g0p_ref[...] = disb * g0_ref[...]


def _tc_scale(d0, d1, g0_pre):
    blk = pl.BlockSpec((BR, D), lambda i: (i, 0))
    dblk = pl.BlockSpec((1, 1, BR), lambda i: (i, 0, 0))
    out = jax.ShapeDtypeStruct((NP, D), jnp.float32)
    return pl.pallas_call(
        _scale_body,
        grid=(RB,),
        in_specs=[dblk, dblk, blk],
        out_specs=[blk, blk],
        out_shape=[out, out],
    )(d0, d1, g0_pre)


def _layer_body(p0_ref, p1_ref, gp_ref, disb_ref, bg_ref, wg_ref, out_ref):
    h = jnp.maximum(
        disb_ref[...] * (p0_ref[...] + p1_ref[...] + 2.0 * gp_ref[...])
        + bg_ref[...], 0.0)
    out_ref[...] = disb_ref[...] * jnp.dot(h, wg_ref[...],
                                           preferred_element_type=jnp.float32)


def _tc_layer(p0, p1, gp, disb, bg2, Wg):
    blk = pl.BlockSpec((BR, D), lambda i: (i, 0))
    full = lambda s: pl.BlockSpec(s, lambda i: (0, 0))
    return pl.pallas_call(
        _layer_body,
        grid=(RB,),
        in_specs=[blk, blk, blk, blk, full((1, D)), full((D, D))],
        out_specs=blk,
        out_shape=jax.ShapeDtypeStruct((NP, D), jnp.float32),
    )(p0, p1, gp, disb, bg2, Wg)


def _final_body(q0_ref, q1_ref, gp_ref, disb_ref, bg_ref, ht_ref,
                wa1_ref, ba1_ref, wa2_ref, ba2_ref, out_ref):
    h2 = jnp.maximum(
        disb_ref[...] * (q0_ref[...] + q1_ref[...] + 2.0 * gp_ref[...])
        + bg_ref[...], 0.0)
    wa_h = wa1_ref[0:D, :] + wa1_ref[D:2 * D, :]
    pre = jnp.maximum(
        jnp.dot(ht_ref[...], wa_h, preferred_element_type=jnp.float32)
        + jnp.dot(h2, wa1_ref[2 * D:3 * D, :], preferred_element_type=jnp.float32)
        + ba1_ref[...], 0.0)
    out_ref[...] = (jnp.dot(pre, wa2_ref[...], preferred_element_type=jnp.float32)
                    + ba2_ref[...])


def _tc_final(q0, q1, g1p, disb, bg2, ht, Wa1, ba12, Wa2p, ba2p):
    blk = pl.BlockSpec((BR, D), lambda i: (i, 0))
    full = lambda s: pl.BlockSpec(s, lambda i: (0, 0))
    return pl.pallas_call(
        _final_body,
        grid=(RB,),
        in_specs=[blk, blk, blk, blk, full((1, D)), blk,
                  full((3 * D, D)), full((1, D)), full((D, D)), full((1, D))],
        out_specs=blk,
        out_shape=jax.ShapeDtypeStruct((NP, D), jnp.float32),
    )(q0, q1, g1p, disb, bg2, ht, Wa1, ba12, Wa2p, ba2p)


def _sc_sim(A, B, row_r, col_r, w2_8x16, b2_16):
    @functools.partial(
        pl.kernel,
        out_type=jax.ShapeDtypeStruct((NW, NCHUNK, K), jnp.float32),
        mesh=_mesh(),
        compiler_params=pltpu.CompilerParams(needs_layout_passes=False),
        scratch_types=[
            pltpu.VMEM((NCHUNK, K), jnp.int32),
            pltpu.VMEM((NCHUNK, K), jnp.int32),
            pltpu.VMEM((K, D), jnp.float32),
            pltpu.VMEM((K, D), jnp.float32),
            pltpu.VMEM((K, D), jnp.float32),
            pltpu.VMEM((K, D), jnp.float32),
            pltpu.VMEM((NCHUNK, K), jnp.float32),
            pltpu.VMEM((8, 16), jnp.float32),
            pltpu.VMEM((16,), jnp.float32),
            pltpu.SemaphoreType.DMA,
            pltpu.SemaphoreType.DMA,
            pltpu.SemaphoreType.DMA,
            pltpu.SemaphoreType.DMA,
        ],
    )
    def k(a_hbm, b_hbm, row_hbm, col_hbm, w2_hbm, b2_hbm, ew_hbm,
          idxr, idxc, abuf0, bbuf0, abuf1, bbuf1, ewbuf, w2v, b2v,
          semA0, semB0, semA1, semB1):
        wid = lax.axis_index("s") * 2 + lax.axis_index("c")
        pltpu.sync_copy(row_hbm.at[wid], idxr)
        pltpu.sync_copy(col_hbm.at[wid], idxc)
        pltpu.sync_copy(w2_hbm, w2v)
        pltpu.sync_copy(b2_hbm, b2v)
        w2l = [_round_bf16(w2v[u, :]) for u in range(8)]
        b2 = b2v[...]
        lanes = lax.iota(jnp.int32, 16)
        abufs = (abuf0, abuf1)
        bbufs = (bbuf0, bbuf1)
        semsA = (semA0, semA1)
        semsB = (semB0, semB1)

        def start(c, bi):
            pltpu.async_copy(a_hbm.at[idxr.at[c]], abufs[bi], semsA[bi])
            pltpu.async_copy(b_hbm.at[idxc.at[c]], bbufs[bi], semsB[bi])

        def wait(c, bi):
            pltpu.make_async_copy(
                a_hbm.at[idxr.at[c]], abufs[bi], semsA[bi]).wait()
            pltpu.make_async_copy(
                b_hbm.at[idxc.at[c]], bbufs[bi], semsB[bi]).wait()

        start(0, 0)
        start(1, 1)

        @pl.loop(0, NCHUNK, step=2)
        def _(c0):
            for b in range(2):
                c = c0 + b

                @pl.when(c < NCHUNK)
                def _():
                    wait(c, b)
                    abuf = abufs[b]
                    bbuf = bbufs[b]

                    def group(g, _):
                        tot = jnp.zeros((16,), jnp.float32)
                        for j16 in range(16):
                            j = g * 16 + j16
                            acc = jnp.zeros((16,), jnp.float32)
                            for u in range(8):
                                av = abuf[j, u * 16:(u + 1) * 16]
                                bv = bbuf[j, u * 16:(u + 1) * 16]
                                z = jnp.maximum(av + bv, 0.0)
                                acc = acc + _round_bf16(z) * w2l[u]
                            tot = jnp.where(lanes == j16, jnp.sum(acc), tot)
                        ewbuf[c, pl.ds(g * 16, 16)] = tot + b2
                        return 0

                    lax.fori_loop(0, 5, group, 0)

                    @pl.when(c + 2 < NCHUNK)
                    def _():
                        start(c + 2, b)

        pltpu.sync_copy(ewbuf, ew_hbm.at[wid])

    return k(A, B, row_r, col_r, w2_8x16, b2_16)


KL = 128
CAP = EW // KL + 2


def _sc_mask_deg(ew, row_r, col_r, thr16):
    @functools.partial(
        pl.kernel,
        out_type=(jax.ShapeDtypeStruct((NW, CAP, KL), jnp.int32),
                  jax.ShapeDtypeStruct((NW, CAP, KL), jnp.int32),
                  jax.ShapeDtypeStruct((NW, CAP, KL), jnp.int32),
                  jax.ShapeDtypeStruct((NW, CAP, KL), jnp.int32),
                  jax.ShapeDtypeStruct((NW, 128), jnp.int32),
                  jax.ShapeDtypeStruct((NW, 128), jnp.int32),
                  jax.ShapeDtypeStruct((2, NP), jnp.float32)),
        mesh=_mesh(),
        compiler_params=pltpu.CompilerParams(needs_layout_passes=False),
        scratch_types=[
            pltpu.VMEM((NCHUNK, K), jnp.float32),
            pltpu.VMEM((NCHUNK, K), jnp.int32),
            pltpu.VMEM((NCHUNK, K), jnp.int32),
            pltpu.VMEM((NCHUNK, K), jnp.float32),
            pltpu.VMEM((CAP * KL,), jnp.int32),
            pltpu.VMEM((CAP * KL,), jnp.int32),
            pltpu.VMEM((CAP * KL,), jnp.int32),
            pltpu.VMEM((CAP * KL,), jnp.int32),
            pltpu.VMEM((16,), jnp.float32),
            pltpu.VMEM((128,), jnp.int32),
            pltpu.VMEM((PT,), jnp.float32),
            pltpu.VMEM_SHARED((NP,), jnp.float32),
        ],
    )
    def k(ew_hbm, row_hbm, col_hbm, thr_hbm,
          crowa_hbm, ccola_hbm, crowb_hbm, ccolb_hbm, cnta_hbm, cntb_hbm,
          degp_hbm,
          ewbuf, rowbuf, colbuf, mbuf, crA, ccA, crB, ccB, thrv, cntbuf,
          zbuf, degacc):
        cid = lax.axis_index("c")
        sid = lax.axis_index("s")
        wid = sid * 2 + cid
        pltpu.sync_copy(ew_hbm.at[wid], ewbuf)
        pltpu.sync_copy(row_hbm.at[wid], rowbuf)
        pltpu.sync_copy(col_hbm.at[wid], colbuf)
        pltpu.sync_copy(thr_hbm, thrv)
        for t in range(PT // 16):
            zbuf[t * 16:(t + 1) * 16] = jnp.zeros((16,), jnp.float32)
        pltpu.sync_copy(zbuf, degacc.at[pl.ds(sid * PT, PT)])
        plsc.subcore_barrier()
        thr = thrv[...]

        def chunk(c, offs):
            offA, offB = offs
            for g in range(K // 16):
                ewv = ewbuf[c, pl.ds(g * 16, 16)]
                cv = colbuf[c, pl.ds(g * 16, 16)]
                rv = rowbuf[c, pl.ds(g * 16, 16)]
                m = ewv > thr
                mbuf[c, pl.ds(g * 16, 16)] = jnp.where(m, 1.0, 0.0)
                mA = m & (cv < SPL)
                mB = m & (cv >= SPL)
                plsc.store_compressed(crA.at[pl.ds(offA, 16)], rv, mask=mA)
                plsc.store_compressed(ccA.at[pl.ds(offA, 16)], cv, mask=mA)
                plsc.store_compressed(crB.at[pl.ds(offB, 16)], rv, mask=mB)
                plsc.store_compressed(ccB.at[pl.ds(offB, 16)], cv - SPL, mask=mB)
                offA = offA + jnp.sum(mA.astype(jnp.int32))
                offB = offB + jnp.sum(mB.astype(jnp.int32))
            pltpu.sync_copy(mbuf.at[c], degacc.at[colbuf.at[c]], add=True)
            return (offA, offB)

        z0 = jnp.zeros((), jnp.int32)
        offA, offB = lax.fori_loop(0, NCHUNK, chunk, (z0, z0))
        zi16 = jnp.zeros((16,), jnp.int32)
        dum16 = jnp.full((16,), DUM, jnp.int32)
        for i in range(KL // 16):
            crA[pl.ds(offA + i * 16, 16)] = zi16
            ccA[pl.ds(offA + i * 16, 16)] = dum16
            crB[pl.ds(offB + i * 16, 16)] = zi16
            ccB[pl.ds(offB + i * 16, 16)] = dum16
        ncA = offA // KL + 1
        ncB = offB // KL + 1
        lanes = lax.iota(jnp.int32, 16)
        for i in range(8):
            cntbuf[i * 16:(i + 1) * 16] = jnp.where(
                lanes == 0, ncA, 0) if i == 0 else jnp.zeros((16,), jnp.int32)
        pltpu.sync_copy(cntbuf, cnta_hbm.at[wid])
        cntbuf[0:16] = jnp.where(lanes == 0, ncB, 0)
        pltpu.sync_copy(cntbuf, cntb_hbm.at[wid])

        def wrA(c, _):
            pltpu.sync_copy(crA.at[pl.ds(c * KL, KL)], crowa_hbm.at[wid, c])
            pltpu.sync_copy(ccA.at[pl.ds(c * KL, KL)], ccola_hbm.at[wid, c])
            return 0

        def wrB(c, _):
            pltpu.sync_copy(crB.at[pl.ds(c * KL, KL)], crowb_hbm.at[wid, c])
            pltpu.sync_copy(ccB.at[pl.ds(c * KL, KL)], ccolb_hbm.at[wid, c])
            return 0

        lax.fori_loop(0, ncA, wrA, 0)
        lax.fori_loop(0, ncB, wrB, 0)
        plsc.subcore_barrier()
        pltpu.sync_copy(degacc.at[pl.ds(sid * PT, PT)],
                        degp_hbm.at[cid, pl.ds(sid * PT, PT)])

    return k(ew, row_r, col_r, thr16)


def _sc_agg_range(gp, crow, ccol, cnt):
    @functools.partial(
        pl.kernel,
        out_type=jax.ShapeDtypeStruct((2, ACR, D), jnp.float32),
        mesh=_mesh(),
        compiler_params=pltpu.CompilerParams(needs_layout_passes=False),
        scratch_types=[
            pltpu.VMEM((CAP, KL), jnp.int32),
            pltpu.VMEM((CAP, KL), jnp.int32),
            pltpu.VMEM((KL, D), jnp.float32),
            pltpu.VMEM((KL, D), jnp.float32),
            pltpu.VMEM((128,), jnp.int32),
            pltpu.VMEM((41, D), jnp.float32),
            pltpu.VMEM_SHARED((ACR, D), jnp.float32),
            pltpu.SemaphoreType.DMA,
            pltpu.SemaphoreType.DMA,
        ],
    )
    def k(g_hbm, crow_hbm, ccol_hbm, cnt_hbm, part_hbm,
          idxr, idxc, gbuf0, gbuf1, cbuf, zbuf, acc, sem0, sem1):
        cid = lax.axis_index("c")
        sid = lax.axis_index("s")
        wid = sid * 2 + cid
        pltpu.sync_copy(cnt_hbm.at[wid], cbuf)
        pltpu.sync_copy(crow_hbm.at[wid], idxr)
        pltpu.sync_copy(ccol_hbm.at[wid], idxc)
        for r in range(41):
            for u in range(8):
                zbuf[r, u * 16:(u + 1) * 16] = jnp.zeros((16,), jnp.float32)
        for b in range(APT // 41):
            pltpu.sync_copy(zbuf, acc.at[pl.ds(sid * APT + b * 41, 41)])
        nc = cbuf[0:16][0]
        plsc.subcore_barrier()
        bufs = (gbuf0, gbuf1)
        sems = (sem0, sem1)

        @pl.when(nc > 0)
        def _():
            pltpu.async_copy(g_hbm.at[idxr.at[0]], gbuf0, sem0)

        @pl.when(nc > 1)
        def _():
            pltpu.async_copy(g_hbm.at[idxr.at[1]], gbuf1, sem1)

        @pl.loop(0, nc, step=2)
        def _(c0):
            for b in range(2):
                c = c0 + b

                @pl.when(c < nc)
                def _():
                    pltpu.make_async_copy(
                        g_hbm.at[idxr.at[c]], bufs[b], sems[b]).wait()
                    pltpu.sync_copy(bufs[b], acc.at[idxc.at[c]], add=True)

                    @pl.when(c + 2 < nc)
                    def _():
                        pltpu.async_copy(
                            g_hbm.at[idxr.at[c + 2]], bufs[b], sems[b])

        plsc.subcore_barrier()
        pltpu.sync_copy(acc.at[pl.ds(sid * APT, APT)],
                        part_hbm.at[cid, pl.ds(sid * APT, APT)])

    return k(gp, crow, ccol, cnt)


def _sc_agg(gp, lists):
    crowA, ccolA, crowB, ccolB, cntA, cntB = lists
    pA = _sc_agg_range(gp, crowA, ccolA, cntA)
    pB = _sc_agg_range(gp, crowB, ccolB, cntB)
    return jnp.concatenate(
        [pA[:, :SPL], pB[:, :NP - SPL]], axis=1)


def kernel(x, edge_index, W_ft, b_ft, W_s1, b_s1, W_s2, b_s2,
           W_g0, b_g0, W_g1, b_g1, W_a1, b_a1, W_a2, b_a2):
    row_r = edge_index[0].reshape(NW, NCHUNK, K)
    col_r = edge_index[1].reshape(NW, NCHUNK, K)
    xp = jnp.pad(x, ((0, NP - N), (0, 0)))

    ht, A, B, g0_pre = _tc_a(xp, W_ft, b_ft.reshape(1, D), W_s1,
                             b_s1.reshape(1, D), W_g0)

    ew = _sc_sim(A, B, row_r, col_r,
                 W_s2[:, 0].reshape(8, 16), jnp.broadcast_to(b_s2, (16,)))

    thr_tile = _tc_thr(ew.reshape(E // 128, 128))
    thr16 = thr_tile[0, :16]

    (crowA, ccolA, crowB, ccolB, cntA, cntB,
     deg_p) = _sc_mask_deg(ew, row_r, col_r, thr16)
    lists = (crowA, ccolA, crowB, ccolB, cntA, cntB)

    d0 = deg_p[0].reshape(RB, 1, BR)
    d1 = deg_p[1].reshape(RB, 1, BR)
    disb, g0p = _tc_scale(d0, d1, g0_pre)

    p = _sc_agg(g0p, lists)
    g1p = _tc_layer(p[0], p[1], g0p, disb, b_g0.reshape(1, D), W_g1)

    q = _sc_agg(g1p, lists)

    Wa2p = jnp.pad(W_a2, ((0, 0), (0, D - 1)))
    ba2p = jnp.pad(b_a2, (0, D - 1)).reshape(1, D)
    scores = _tc_final(q[0], q[1], g1p, disb, b_g1.reshape(1, D), ht,
                       W_a1, b_a1.reshape(1, D), Wa2p, ba2p)
    return scores[:N, 0]


# speedup vs baseline: 13.2345x; 1.0126x over previous
import functools

import jax
import jax.numpy as jnp
from jax import lax
from jax.experimental import pallas as pl
from jax.experimental.pallas import tpu as pltpu
from jax.experimental.pallas import tpu_sc as plsc

N = 10000
E = 320000
D = 128
NP = 10240
NW = 32
EW = E // NW
K = 80
NCHUNK = EW // K
PT = NP // 16
SPL = 5120
ACR = 5248
APT = ACR // 16
DUM = ACR - 1
RB = 20
BR = NP // RB


def _mesh():
    return plsc.VectorSubcoreMesh(core_axis_name="c", subcore_axis_name="s")


def _round_bf16(z):
    u = plsc.bitcast(z, jnp.uint32)
    r = (u + jnp.uint32(0x7FFF) + ((u >> jnp.uint32(16)) & jnp.uint32(1))) \
        & jnp.uint32(0xFFFF0000)
    return plsc.bitcast(r, jnp.float32)


def _tca_body(x_ref, wft_ref, bft_ref, ws1_ref, bs1_ref, wg0_ref,
              ht_ref, a_ref, b_ref, g0_ref):
    ht = jnp.maximum(
        jnp.dot(x_ref[...], wft_ref[...], preferred_element_type=jnp.float32)
        + bft_ref[...], 0.0)
    ht_ref[...] = ht
    a_ref[...] = jnp.dot(ht, ws1_ref[0:D, :], preferred_element_type=jnp.float32)
    b_ref[...] = (jnp.dot(ht, ws1_ref[D:2 * D, :], preferred_element_type=jnp.float32)
                  + bs1_ref[...])
    g0_ref[...] = jnp.dot(ht, wg0_ref[...], preferred_element_type=jnp.float32)


def _tc_a(xp, W_ft, b_ft2, W_s1, b_s12, W_g0):
    blk = pl.BlockSpec((BR, D), lambda i: (i, 0))
    full = lambda s: pl.BlockSpec(s, lambda i: (0, 0))
    out = jax.ShapeDtypeStruct((NP, D), jnp.float32)
    return pl.pallas_call(
        _tca_body,
        grid=(RB,),
        in_specs=[blk, full((D, D)), full((1, D)), full((2 * D, D)),
                  full((1, D)), full((D, D))],
        out_specs=[blk, blk, blk, blk],
        out_shape=[out, out, out, out],
    )(xp, W_ft, b_ft2, W_s1, b_s12, W_g0)


def _thr_body(t_ref, out_ref):
    v = 1.0 / (1.0 + jnp.exp(-t_ref[...]))
    s1 = jnp.sum(v)
    s2 = jnp.sum(v * v)
    m = s1 / E
    var = (s2 - s1 * s1 / E) / (E - 1)
    thr = m + jnp.sqrt(var)
    tl = jnp.where(thr < 1.0, jnp.log(thr / (1.0 - thr)), jnp.float32(3.0e38))
    out_ref[...] = jnp.full((8, 128), tl, jnp.float32)


def _tc_thr(ew2d):
    return pl.pallas_call(
        _thr_body,
        out_shape=jax.ShapeDtypeStruct((8, 128), jnp.float32),
    )(ew2d)


def _scale_body(d0_ref, d1_ref, g0_ref, disb_ref, g0p_ref):
    d = d0_ref[0, 0, :] + d1_ref[0, 0, :] + 2.0
    dis = lax.rsqrt(d)
    disb = jnp.broadcast_to(dis[:, None], (BR, D))
    disb_ref[...] = disb
    g0p_ref[...] = disb * g0_ref[...]


def _tc_scale(d0, d1, g0_pre):
    blk = pl.BlockSpec((BR, D), lambda i: (i, 0))
    dblk = pl.BlockSpec((1, 1, BR), lambda i: (i, 0, 0))
    out = jax.ShapeDtypeStruct((NP, D), jnp.float32)
    return pl.pallas_call(
        _scale_body,
        grid=(RB,),
        in_specs=[dblk, dblk, blk],
        out_specs=[blk, blk],
        out_shape=[out, out],
    )(d0, d1, g0_pre)


def _layer_body(p0_ref, p1_ref, gp_ref, disb_ref, bg_ref, wg_ref, out_ref):
    h = jnp.maximum(
        disb_ref[...] * (p0_ref[...] + p1_ref[...] + 2.0 * gp_ref[...])
        + bg_ref[...], 0.0)
    out_ref[...] = disb_ref[...] * jnp.dot(h, wg_ref[...],
                                           preferred_element_type=jnp.float32)


def _tc_layer(p0, p1, gp, disb, bg2, Wg):
    blk = pl.BlockSpec((BR, D), lambda i: (i, 0))
    full = lambda s: pl.BlockSpec(s, lambda i: (0, 0))
    return pl.pallas_call(
        _layer_body,
        grid=(RB,),
        in_specs=[blk, blk, blk, blk, full((1, D)), full((D, D))],
        out_specs=blk,
        out_shape=jax.ShapeDtypeStruct((NP, D), jnp.float32),
    )(p0, p1, gp, disb, bg2, Wg)


def _final_body(q0_ref, q1_ref, gp_ref, disb_ref, bg_ref, ht_ref,
                wa1_ref, ba1_ref, wa2_ref, ba2_ref, out_ref):
    h2 = jnp.maximum(
        disb_ref[...] * (q0_ref[...] + q1_ref[...] + 2.0 * gp_ref[...])
        + bg_ref[...], 0.0)
    wa_h = wa1_ref[0:D, :] + wa1_ref[D:2 * D, :]
    pre = jnp.maximum(
        jnp.dot(ht_ref[...], wa_h, preferred_element_type=jnp.float32)
        + jnp.dot(h2, wa1_ref[2 * D:3 * D, :], preferred_element_type=jnp.float32)
        + ba1_ref[...], 0.0)
    out_ref[...] = (jnp.dot(pre, wa2_ref[...], preferred_element_type=jnp.float32)
                    + ba2_ref[...])


def _tc_final(q0, q1, g1p, disb, bg2, ht, Wa1, ba12, Wa2p, ba2p):
    blk = pl.BlockSpec((BR, D), lambda i: (i, 0))
    full = lambda s: pl.BlockSpec(s, lambda i: (0, 0))
    return pl.pallas_call(
        _final_body,
        grid=(RB,),
        in_specs=[blk, blk, blk, blk, full((1, D)), blk,
                  full((3 * D, D)), full((1, D)), full((D, D)), full((1, D))],
        out_specs=blk,
        out_shape=jax.ShapeDtypeStruct((NP, D), jnp.float32),
    )(q0, q1, g1p, disb, bg2, ht, Wa1, ba12, Wa2p, ba2p)


def _sc_sim(A, B, row_r, col_r, w2_8x16, b2_16):
    @functools.partial(
        pl.kernel,
        out_type=jax.ShapeDtypeStruct((NW, NCHUNK, K), jnp.float32),
        mesh=_mesh(),
        compiler_params=pltpu.CompilerParams(needs_layout_passes=False),
        scratch_types=[
            pltpu.VMEM((NCHUNK, K), jnp.int32),
            pltpu.VMEM((NCHUNK, K), jnp.int32),
            pltpu.VMEM((K, D), jnp.float32),
            pltpu.VMEM((K, D), jnp.float32),
            pltpu.VMEM((K, D), jnp.float32),
            pltpu.VMEM((K, D), jnp.float32),
            pltpu.VMEM((NCHUNK, K), jnp.float32),
            pltpu.VMEM((8, 16), jnp.float32),
            pltpu.VMEM((16,), jnp.float32),
            pltpu.SemaphoreType.DMA,
            pltpu.SemaphoreType.DMA,
            pltpu.SemaphoreType.DMA,
            pltpu.SemaphoreType.DMA,
        ],
    )
    def k(a_hbm, b_hbm, row_hbm, col_hbm, w2_hbm, b2_hbm, ew_hbm,
          idxr, idxc, abuf0, bbuf0, abuf1, bbuf1, ewbuf, w2v, b2v,
          semA0, semB0, semA1, semB1):
        wid = lax.axis_index("s") * 2 + lax.axis_index("c")
        pltpu.sync_copy(row_hbm.at[wid], idxr)
        pltpu.sync_copy(col_hbm.at[wid], idxc)
        pltpu.sync_copy(w2_hbm, w2v)
        pltpu.sync_copy(b2_hbm, b2v)
        w2l = [_round_bf16(w2v[u, :]) for u in range(8)]
        b2 = b2v[...]
        lanes = lax.iota(jnp.int32, 16)
        abufs = (abuf0, abuf1)
        bbufs = (bbuf0, bbuf1)
        semsA = (semA0, semA1)
        semsB = (semB0, semB1)

        def start(c, bi):
            pltpu.async_copy(a_hbm.at[idxr.at[c]], abufs[bi], semsA[bi])
            pltpu.async_copy(b_hbm.at[idxc.at[c]], bbufs[bi], semsB[bi])

        def wait(c, bi):
            pltpu.make_async_copy(
                a_hbm.at[idxr.at[c]], abufs[bi], semsA[bi]).wait()
            pltpu.make_async_copy(
                b_hbm.at[idxc.at[c]], bbufs[bi], semsB[bi]).wait()

        start(0, 0)
        start(1, 1)

        @pl.loop(0, NCHUNK, step=2)
        def _(c0):
            for b in range(2):
                c = c0 + b

                @pl.when(c < NCHUNK)
                def _():
                    wait(c, b)
                    abuf = abufs[b]
                    bbuf = bbufs[b]

                    def group(g, _):
                        tot = jnp.zeros((16,), jnp.float32)
                        for j16 in range(16):
                            j = g * 16 + j16
                            acc = jnp.zeros((16,), jnp.float32)
                            for u in range(8):
                                av = abuf[j, u * 16:(u + 1) * 16]
                                bv = bbuf[j, u * 16:(u + 1) * 16]
                                z = jnp.maximum(av + bv, 0.0)
                                acc = acc + _round_bf16(z) * w2l[u]
                            tot = jnp.where(lanes == j16, jnp.sum(acc), tot)
                        ewbuf[c, pl.ds(g * 16, 16)] = tot + b2
                        return 0

                    lax.fori_loop(0, 5, group, 0)

                    @pl.when(c + 2 < NCHUNK)
                    def _():
                        start(c + 2, b)

        pltpu.sync_copy(ewbuf, ew_hbm.at[wid])

    return k(A, B, row_r, col_r, w2_8x16, b2_16)


KL = 128
CAP = EW // KL + 2


def _sc_mask_deg(ew, row_r, col_r, thr16):
    @functools.partial(
        pl.kernel,
        out_type=(jax.ShapeDtypeStruct((NW, CAP, KL), jnp.int32),
                  jax.ShapeDtypeStruct((NW, CAP, KL), jnp.int32),
                  jax.ShapeDtypeStruct((NW, CAP, KL), jnp.int32),
                  jax.ShapeDtypeStruct((NW, CAP, KL), jnp.int32),
                  jax.ShapeDtypeStruct((NW, 128), jnp.int32),
                  jax.ShapeDtypeStruct((NW, 128), jnp.int32),
                  jax.ShapeDtypeStruct((2, NP), jnp.float32)),
        mesh=_mesh(),
        compiler_params=pltpu.CompilerParams(needs_layout_passes=False),
        scratch_types=[
            pltpu.VMEM((NCHUNK, K), jnp.float32),
            pltpu.VMEM((NCHUNK, K), jnp.int32),
            pltpu.VMEM((NCHUNK, K), jnp.int32),
            pltpu.VMEM((NCHUNK, K), jnp.float32),
            pltpu.VMEM((CAP * KL,), jnp.int32),
            pltpu.VMEM((CAP * KL,), jnp.int32),
            pltpu.VMEM((CAP * KL,), jnp.int32),
            pltpu.VMEM((CAP * KL,), jnp.int32),
            pltpu.VMEM((16,), jnp.float32),
            pltpu.VMEM((128,), jnp.int32),
            pltpu.VMEM((PT,), jnp.float32),
            pltpu.VMEM_SHARED((NP,), jnp.float32),
        ],
    )
    def k(ew_hbm, row_hbm, col_hbm, thr_hbm,
          crowa_hbm, ccola_hbm, crowb_hbm, ccolb_hbm, cnta_hbm, cntb_hbm,
          degp_hbm,
          ewbuf, rowbuf, colbuf, mbuf, crA, ccA, crB, ccB, thrv, cntbuf,
          zbuf, degacc):
        cid = lax.axis_index("c")
        sid = lax.axis_index("s")
        wid = sid * 2 + cid
        pltpu.sync_copy(ew_hbm.at[wid], ewbuf)
        pltpu.sync_copy(row_hbm.at[wid], rowbuf)
        pltpu.sync_copy(col_hbm.at[wid], colbuf)
        pltpu.sync_copy(thr_hbm, thrv)
        for t in range(PT // 16):
            zbuf[t * 16:(t + 1) * 16] = jnp.zeros((16,), jnp.float32)
        pltpu.sync_copy(zbuf, degacc.at[pl.ds(sid * PT, PT)])
        plsc.subcore_barrier()
        thr = thrv[...]

        def chunk(c, offs):
            offA, offB = offs
            for g in range(K // 16):
                ewv = ewbuf[c, pl.ds(g * 16, 16)]
                cv = colbuf[c, pl.ds(g * 16, 16)]
                rv = rowbuf[c, pl.ds(g * 16, 16)]
                m = ewv > thr
                mbuf[c, pl.ds(g * 16, 16)] = jnp.where(m, 1.0, 0.0)
                mA = m & (cv < SPL)
                mB = m & (cv >= SPL)
                plsc.store_compressed(crA.at[pl.ds(offA, 16)], rv, mask=mA)
                plsc.store_compressed(ccA.at[pl.ds(offA, 16)], cv, mask=mA)
                plsc.store_compressed(crB.at[pl.ds(offB, 16)], rv, mask=mB)
                plsc.store_compressed(ccB.at[pl.ds(offB, 16)], cv - SPL, mask=mB)
                offA = offA + jnp.sum(mA.astype(jnp.int32))
                offB = offB + jnp.sum(mB.astype(jnp.int32))
            pltpu.sync_copy(mbuf.at[c], degacc.at[colbuf.at[c]], add=True)
            return (offA, offB)

        z0 = jnp.zeros((), jnp.int32)
        offA, offB = lax.fori_loop(0, NCHUNK, chunk, (z0, z0))
        zi16 = jnp.zeros((16,), jnp.int32)
        dum16 = jnp.full((16,), DUM, jnp.int32)
        for i in range(KL // 16):
            crA[pl.ds(offA + i * 16, 16)] = zi16
            ccA[pl.ds(offA + i * 16, 16)] = dum16
            crB[pl.ds(offB + i * 16, 16)] = zi16
            ccB[pl.ds(offB + i * 16, 16)] = dum16
        ncA = offA // KL + 1
        ncB = offB // KL + 1
        lanes = lax.iota(jnp.int32, 16)
        for i in range(8):
            cntbuf[i * 16:(i + 1) * 16] = jnp.where(
                lanes == 0, ncA, 0) if i == 0 else jnp.zeros((16,), jnp.int32)
        pltpu.sync_copy(cntbuf, cnta_hbm.at[wid])
        cntbuf[0:16] = jnp.where(lanes == 0, ncB, 0)
        pltpu.sync_copy(cntbuf, cntb_hbm.at[wid])

        def wrA(c, _):
            pltpu.sync_copy(crA.at[pl.ds(c * KL, KL)], crowa_hbm.at[wid, c])
            pltpu.sync_copy(ccA.at[pl.ds(c * KL, KL)], ccola_hbm.at[wid, c])
            return 0

        def wrB(c, _):
            pltpu.sync_copy(crB.at[pl.ds(c * KL, KL)], crowb_hbm.at[wid, c])
            pltpu.sync_copy(ccB.at[pl.ds(c * KL, KL)], ccolb_hbm.at[wid, c])
            return 0

        lax.fori_loop(0, ncA, wrA, 0)
        lax.fori_loop(0, ncB, wrB, 0)
        plsc.subcore_barrier()
        pltpu.sync_copy(degacc.at[pl.ds(sid * PT, PT)],
                        degp_hbm.at[cid, pl.ds(sid * PT, PT)])

    return k(ew, row_r, col_r, thr16)


def _sc_agg_layer(gp, crowA, ccolA, crowB, ccolB, cntA, cntB):
    @functools.partial(
        pl.kernel,
        out_type=(jax.ShapeDtypeStruct((2, ACR, D), jnp.float32),
                  jax.ShapeDtypeStruct((2, ACR, D), jnp.float32)),
        mesh=_mesh(),
        compiler_params=pltpu.CompilerParams(needs_layout_passes=False),
        scratch_types=[
            pltpu.VMEM((CAP, KL), jnp.int32),
            pltpu.VMEM((CAP, KL), jnp.int32),
            pltpu.VMEM((CAP, KL), jnp.int32),
            pltpu.VMEM((CAP, KL), jnp.int32),
            pltpu.VMEM((KL, D), jnp.float32),
            pltpu.VMEM((KL, D), jnp.float32),
            pltpu.VMEM((128,), jnp.int32),
            pltpu.VMEM((41, D), jnp.float32),
            pltpu.VMEM_SHARED((ACR, D), jnp.float32),
            pltpu.SemaphoreType.DMA,
            pltpu.SemaphoreType.DMA,
        ],
    )
    def k(g_hbm, crowa_hbm, ccola_hbm, crowb_hbm, ccolb_hbm,
          cnta_hbm, cntb_hbm, parta_hbm, partb_hbm,
          idxrA, idxcA, idxrB, idxcB, gbuf0, gbuf1, cbuf, zbuf, acc,
          sem0, sem1):
        cid = lax.axis_index("c")
        sid = lax.axis_index("s")
        wid = sid * 2 + cid
        pltpu.sync_copy(crowa_hbm.at[wid], idxrA)
        pltpu.sync_copy(ccola_hbm.at[wid], idxcA)
        pltpu.sync_copy(crowb_hbm.at[wid], idxrB)
        pltpu.sync_copy(ccolb_hbm.at[wid], idxcB)
        pltpu.sync_copy(cnta_hbm.at[wid], cbuf)
        ncA = cbuf[0:16][0]
        pltpu.sync_copy(cntb_hbm.at[wid], cbuf)
        ncB = cbuf[0:16][0]
        for r in range(41):
            for u in range(8):
                zbuf[r, u * 16:(u + 1) * 16] = jnp.zeros((16,), jnp.float32)
        bufs = (gbuf0, gbuf1)
        sems = (sem0, sem1)

        def run_range(idxr, idxc, nc, part_hbm):
            for b in range(APT // 41):
                pltpu.sync_copy(zbuf, acc.at[pl.ds(sid * APT + b * 41, 41)])
            plsc.subcore_barrier()

            @pl.when(nc > 0)
            def _():
                pltpu.async_copy(g_hbm.at[idxr.at[0]], gbuf0, sem0)

            @pl.when(nc > 1)
            def _():
                pltpu.async_copy(g_hbm.at[idxr.at[1]], gbuf1, sem1)

            @pl.loop(0, nc, step=2)
            def _(c0):
                for b in range(2):
                    c = c0 + b

                    @pl.when(c < nc)
                    def _():
                        pltpu.make_async_copy(
                            g_hbm.at[idxr.at[c]], bufs[b], sems[b]).wait()
                        pltpu.sync_copy(bufs[b], acc.at[idxc.at[c]], add=True)

                        @pl.when(c + 2 < nc)
                        def _():
                            pltpu.async_copy(
                                g_hbm.at[idxr.at[c + 2]], bufs[b], sems[b])

            plsc.subcore_barrier()
            pltpu.sync_copy(acc.at[pl.ds(sid * APT, APT)],
                            part_hbm.at[cid, pl.ds(sid * APT, APT)])
            plsc.subcore_barrier()

        run_range(idxrA, idxcA, ncA, parta_hbm)
        run_range(idxrB, idxcB, ncB, partb_hbm)

    return k(gp, crowA, ccolA, crowB, ccolB, cntA, cntB)


def _sc_agg(gp, lists):
    crowA, ccolA, crowB, ccolB, cntA, cntB = lists
    pA, pB = _sc_agg_layer(gp, crowA, ccolA, crowB, ccolB, cntA, cntB)
    return jnp.concatenate(
        [pA[:, :SPL], pB[:, :NP - SPL]], axis=1)


def kernel(x, edge_index, W_ft, b_ft, W_s1, b_s1, W_s2, b_s2,
           W_g0, b_g0, W_g1, b_g1, W_a1, b_a1, W_a2, b_a2):
    row_r = edge_index[0].reshape(NW, NCHUNK, K)
    col_r = edge_index[1].reshape(NW, NCHUNK, K)
    xp = jnp.pad(x, ((0, NP - N), (0, 0)))

    ht, A, B, g0_pre = _tc_a(xp, W_ft, b_ft.reshape(1, D), W_s1,
                             b_s1.reshape(1, D), W_g0)

    ew = _sc_sim(A, B, row_r, col_r,
                 W_s2[:, 0].reshape(8, 16), jnp.broadcast_to(b_s2, (16,)))

    thr_tile = _tc_thr(ew.reshape(E // 128, 128))
    thr16 = thr_tile[0, :16]

    (crowA, ccolA, crowB, ccolB, cntA, cntB,
     deg_p) = _sc_mask_deg(ew, row_r, col_r, thr16)
    lists = (crowA, ccolA, crowB, ccolB, cntA, cntB)

    d0 = deg_p[0].reshape(RB, 1, BR)
    d1 = deg_p[1].reshape(RB, 1, BR)
    disb, g0p = _tc_scale(d0, d1, g0_pre)

    p = _sc_agg(g0p, lists)
    g1p = _tc_layer(p[0], p[1], g0p, disb, b_g0.reshape(1, D), W_g1)

    q = _sc_agg(g1p, lists)

    Wa2p = jnp.pad(W_a2, ((0, 0), (0, D - 1)))
    ba2p = jnp.pad(b_a2, (0, D - 1)).reshape(1, D)
    scores = _tc_final(q[0], q[1], g1p, disb, b_g1.reshape(1, D), ht,
                       W_a1, b_a1.reshape(1, D), Wa2p, ba2p)
    return scores[:N, 0]
